# baseline (speedup 1.0000x reference)
import functools

import jax
import jax.numpy as jnp
from jax import lax
from jax.experimental import pallas as pl
from jax.experimental.pallas import tpu as pltpu
from jax.experimental.pallas import tpu_sc as plsc

VOCAB = 1000000
CTX = 2048
D = 64

NUM_CORES = 2
NUM_SUBCORES = 16
NUM_WORKERS = NUM_CORES * NUM_SUBCORES


def _sinusoidal_pe(context_size, embedding_size):
    positions = jnp.arange(context_size, dtype=jnp.float32)
    indices = jnp.arange(embedding_size // 2, dtype=jnp.float32)
    scaling_factor = 10000.0 ** (2.0 * indices / embedding_size)
    angles = positions[:, None] / scaling_factor
    pe = jnp.zeros((context_size, embedding_size), dtype=jnp.float32)
    pe = pe.at[:, 0::2].set(jnp.sin(angles))
    pe = pe.at[:, 1::2].set(jnp.cos(angles))
    return pe


def _make_sc_kernel(batch, rows_per_worker):
    mesh = plsc.VectorSubcoreMesh(
        core_axis_name="c",
        subcore_axis_name="s",
        num_cores=NUM_CORES,
        num_subcores=NUM_SUBCORES,
    )

    @functools.partial(
        pl.kernel,
        out_type=jax.ShapeDtypeStruct((batch, D), jnp.float32),
        mesh=mesh,
        scratch_types=[
            pltpu.VMEM((rows_per_worker,), jnp.int32),
            pltpu.VMEM((rows_per_worker, D), jnp.float32),
            pltpu.SemaphoreType.DMA,
        ],
        compiler_params=pltpu.CompilerParams(use_tc_tiling_on_sc=False),
    )
    def body(idx_hbm, pe_hbm, table_hbm, out_hbm, idx_v, rows_v, sem):
        wid = lax.axis_index("s") * NUM_CORES + lax.axis_index("c")
        base = wid * rows_per_worker
        pe_base = lax.rem(base, CTX)
        pltpu.sync_copy(idx_hbm.at[pl.ds(base, rows_per_worker)], idx_v)
        pltpu.sync_copy(pe_hbm.at[pl.ds(pe_base, rows_per_worker)], rows_v)
        pltpu.async_copy(table_hbm.at[idx_v], rows_v, sem, add=True).wait()
        pltpu.sync_copy(rows_v, out_hbm.at[pl.ds(base, rows_per_worker)])

    return body


def kernel(x, table):
    pe = _sinusoidal_pe(CTX, D)
    idx = x.reshape(-1).astype(jnp.int32)
    batch = idx.shape[0]
    rows_per_worker = batch // NUM_WORKERS
    out = _make_sc_kernel(batch, rows_per_worker)(idx, pe, table)
    return out.reshape(x.shape + (D,))


# speedup vs baseline: 1.6865x; 1.6865x over previous
import functools

import jax
import jax.numpy as jnp
from jax import lax
from jax.experimental import pallas as pl
from jax.experimental.pallas import tpu as pltpu
from jax.experimental.pallas import tpu_sc as plsc

VOCAB = 1000000
CTX = 2048
D = 64

NUM_CORES = 2
NUM_SUBCORES = 16
NUM_WORKERS = NUM_CORES * NUM_SUBCORES


def _sinusoidal_pe(context_size, embedding_size):
    positions = jnp.arange(context_size, dtype=jnp.float32)
    indices = jnp.arange(embedding_size // 2, dtype=jnp.float32)
    scaling_factor = 10000.0 ** (2.0 * indices / embedding_size)
    angles = positions[:, None] / scaling_factor
    pe = jnp.zeros((context_size, embedding_size), dtype=jnp.float32)
    pe = pe.at[:, 0::2].set(jnp.sin(angles))
    pe = pe.at[:, 1::2].set(jnp.cos(angles))
    return pe


def _make_sc_kernel(batch, rows_per_worker):
    mesh = plsc.VectorSubcoreMesh(
        core_axis_name="c",
        subcore_axis_name="s",
        num_cores=NUM_CORES,
        num_subcores=NUM_SUBCORES,
    )

    @functools.partial(
        pl.kernel,
        out_type=jax.ShapeDtypeStruct((batch, D), jnp.float32),
        mesh=mesh,
        scratch_types=[
            pltpu.VMEM((rows_per_worker,), jnp.int32),
            pltpu.VMEM((rows_per_worker, D), jnp.float32),
            pltpu.VMEM((rows_per_worker, D), jnp.float32),
            pltpu.SemaphoreType.DMA,
        ],
    )
    def body(idx_hbm, pe_hbm, table_hbm, out_hbm, idx_s, rows_v, pe_v, sem):
        wid = lax.axis_index("s") * NUM_CORES + lax.axis_index("c")
        base = wid * rows_per_worker
        pe_base = lax.rem(base, CTX)
        pltpu.sync_copy(idx_hbm.at[pl.ds(base, rows_per_worker)], idx_s)
        pe_cp = pltpu.async_copy(
            pe_hbm.at[pl.ds(pe_base, rows_per_worker)], pe_v, sem
        )

        def fire_chunk(ch, _):
            vec = idx_s[pl.ds(ch * 16, 16)]
            for j in range(16):
                pltpu.async_copy(
                    table_hbm.at[pl.ds(vec[j], 1)],
                    rows_v.at[pl.ds(ch * 16 + j, 1)],
                    sem,
                )
            return 0

        lax.fori_loop(0, rows_per_worker // 16, fire_chunk, 0)
        pe_cp.wait()

        def drain_row(r, _):
            pltpu.make_async_copy(
                table_hbm.at[pl.ds(0, 1)], rows_v.at[pl.ds(r, 1)], sem
            ).wait()
            return 0

        lax.fori_loop(0, rows_per_worker, drain_row, 0)

        def add_row(r, _):
            for c in range(D // 16):
                sl = pl.ds(c * 16, 16)
                rows_v[r, sl] = rows_v[r, sl] + pe_v[r, sl]
            return 0

        lax.fori_loop(0, rows_per_worker, add_row, 0)
        pltpu.sync_copy(rows_v, out_hbm.at[pl.ds(base, rows_per_worker)])

    return body


def kernel(x, table):
    pe = _sinusoidal_pe(CTX, D)
    idx = x.reshape(-1).astype(jnp.int32)
    batch = idx.shape[0]
    rows_per_worker = batch // NUM_WORKERS
    out = _make_sc_kernel(batch, rows_per_worker)(idx, pe, table)
    return out.reshape(x.shape + (D,))


# speedup vs baseline: 1.6869x; 1.0002x over previous
import functools

import jax
import jax.numpy as jnp
from jax import lax
from jax.experimental import pallas as pl
from jax.experimental.pallas import tpu as pltpu
from jax.experimental.pallas import tpu_sc as plsc

VOCAB = 1000000
CTX = 2048
D = 64

NUM_CORES = 2
NUM_SUBCORES = 16
NUM_WORKERS = NUM_CORES * NUM_SUBCORES


def _sinusoidal_pe(context_size, embedding_size):
    positions = jnp.arange(context_size, dtype=jnp.float32)
    indices = jnp.arange(embedding_size // 2, dtype=jnp.float32)
    scaling_factor = 10000.0 ** (2.0 * indices / embedding_size)
    angles = positions[:, None] / scaling_factor
    pe = jnp.zeros((context_size, embedding_size), dtype=jnp.float32)
    pe = pe.at[:, 0::2].set(jnp.sin(angles))
    pe = pe.at[:, 1::2].set(jnp.cos(angles))
    return pe


def _make_sc_kernel(batch, rows_per_worker):
    mesh = plsc.VectorSubcoreMesh(
        core_axis_name="c",
        subcore_axis_name="s",
        num_cores=NUM_CORES,
        num_subcores=NUM_SUBCORES,
    )

    @functools.partial(
        pl.kernel,
        out_type=jax.ShapeDtypeStruct((batch, D), jnp.float32),
        mesh=mesh,
        scratch_types=[
            pltpu.VMEM((rows_per_worker,), jnp.int32),
            pltpu.VMEM((rows_per_worker, D), jnp.float32),
            pltpu.VMEM((rows_per_worker, D), jnp.float32),
            pltpu.SemaphoreType.DMA,
        ],
    )
    def body(idx_hbm, pe_hbm, table_hbm, out_hbm, idx_s, rows_v, pe_v, sem):
        wid = lax.axis_index("s") * NUM_CORES + lax.axis_index("c")
        base = wid * rows_per_worker
        pe_base = lax.rem(base, CTX)
        pltpu.sync_copy(idx_hbm.at[pl.ds(base, rows_per_worker)], idx_s)
        pe_cp = pltpu.async_copy(
            pe_hbm.at[pl.ds(pe_base, rows_per_worker)], pe_v, sem
        )

        with jax.named_scope("fire_gathers"):
            def fire_chunk(ch, _):
                vec = idx_s[pl.ds(ch * 16, 16)]
                for j in range(16):
                    pltpu.async_copy(
                        table_hbm.at[pl.ds(vec[j], 1)],
                        rows_v.at[pl.ds(ch * 16 + j, 1)],
                        sem,
                    )
                return 0

            lax.fori_loop(0, rows_per_worker // 16, fire_chunk, 0)
        pe_cp.wait()

        with jax.named_scope("drain_gathers"):
            def drain_row(r, _):
                pltpu.make_async_copy(
                    table_hbm.at[pl.ds(0, 1)], rows_v.at[pl.ds(r, 1)], sem
                ).wait()
                return 0

            lax.fori_loop(0, rows_per_worker, drain_row, 0)

        with jax.named_scope("pe_add"):
            def add_row(r, _):
                for c in range(D // 16):
                    sl = pl.ds(c * 16, 16)
                    rows_v[r, sl] = rows_v[r, sl] + pe_v[r, sl]
                return 0

            lax.fori_loop(0, rows_per_worker, add_row, 0)
        with jax.named_scope("store_out"):
            pltpu.sync_copy(rows_v, out_hbm.at[pl.ds(base, rows_per_worker)])

    return body


def kernel(x, table):
    pe = _sinusoidal_pe(CTX, D)
    idx = x.reshape(-1).astype(jnp.int32)
    batch = idx.shape[0]
    rows_per_worker = batch // NUM_WORKERS
    out = _make_sc_kernel(batch, rows_per_worker)(idx, pe, table)
    return out.reshape(x.shape + (D,))


# speedup vs baseline: 3.5262x; 2.0903x over previous
import functools

import jax
import jax.numpy as jnp
from jax import lax
from jax.experimental import pallas as pl
from jax.experimental.pallas import tpu as pltpu
from jax.experimental.pallas import tpu_sc as plsc

VOCAB = 1000000
CTX = 2048
D = 64

NUM_CORES = 2
NUM_SUBCORES = 16
NUM_WORKERS = NUM_CORES * NUM_SUBCORES
NBUF = 4
EDGE = (VOCAB // 128) * 128


def _sinusoidal_pe(context_size, embedding_size):
    positions = jnp.arange(context_size, dtype=jnp.float32)
    indices = jnp.arange(embedding_size // 2, dtype=jnp.float32)
    scaling_factor = 10000.0 ** (2.0 * indices / embedding_size)
    angles = positions[:, None] / scaling_factor
    pe = jnp.zeros((context_size, embedding_size), dtype=jnp.float32)
    pe = pe.at[:, 0::2].set(jnp.sin(angles))
    pe = pe.at[:, 1::2].set(jnp.cos(angles))
    return pe


def _make_sc_kernel(batch, tok_per_worker):
    mesh = plsc.VectorSubcoreMesh(
        core_axis_name="c",
        subcore_axis_name="s",
        num_cores=NUM_CORES,
        num_subcores=NUM_SUBCORES,
    )

    @functools.partial(
        pl.kernel,
        out_type=jax.ShapeDtypeStruct((batch, D), jnp.float32),
        mesh=mesh,
        scratch_types=[
            pltpu.VMEM((tok_per_worker + 16,), jnp.int32),
            pltpu.VMEM((tok_per_worker, D), jnp.float32),
            pltpu.VMEM((tok_per_worker, D), jnp.float32),
            pltpu.VMEM((NBUF + 1, D, 128), jnp.float32),
            pltpu.SemaphoreType.DMA,
            pltpu.SemaphoreType.DMA,
        ],
        compiler_params=pltpu.CompilerParams(needs_layout_passes=False),
    )
    def body(
        idx_hbm, pe_hbm, tableT_hbm, tail_hbm, out_hbm,
        idx_s, rows_v, pe_v, bufs, sem, sem_pe,
    ):
        wid = lax.axis_index("s") * NUM_CORES + lax.axis_index("c")
        base = wid * tok_per_worker
        pe_base = lax.rem(base, CTX)
        pltpu.sync_copy(
            idx_hbm.at[pl.ds(base, tok_per_worker)],
            idx_s.at[pl.ds(0, tok_per_worker)],
        )
        pltpu.sync_copy(tail_hbm, bufs.at[NBUF])
        pe_cp = pltpu.async_copy(
            pe_hbm.at[pl.ds(pe_base, tok_per_worker)], pe_v, sem_pe
        )
        iota16 = lax.iota(jnp.int32, 16)

        def do_chunk(ch, _):
            vec = idx_s[pl.ds(ch * NBUF, 16)]
            offs = [
                jnp.minimum((vec[j] // 128) * 128, EDGE - 128)
                for j in range(NBUF)
            ]
            for j in range(NBUF):
                pltpu.async_copy(
                    tableT_hbm.at[:, pl.ds(offs[j], 128)], bufs.at[j], sem
                )
            for j in range(NBUF):
                pltpu.make_async_copy(
                    tableT_hbm.at[:, pl.ds(0, 128)], bufs.at[j], sem
                ).wait()
            for j in range(NBUF):
                edge = vec[j] >= EDGE
                sel = jnp.full(
                    (16,), jnp.where(edge, NBUF, j), jnp.int32
                )
                col = jnp.where(edge, vec[j] - EDGE, vec[j] - offs[j])
                voff = jnp.full((16,), col, jnp.int32)
                r = ch * NBUF + j
                for c in range(D // 16):
                    g = plsc.load_gather(
                        bufs, [sel, iota16 + (16 * c), voff]
                    )
                    rows_v[r, pl.ds(16 * c, 16)] = g
            return 0

        lax.fori_loop(0, tok_per_worker // NBUF, do_chunk, 0)
        pe_cp.wait()

        def add_row(r, _):
            for c in range(D // 16):
                sl = pl.ds(c * 16, 16)
                rows_v[r, sl] = rows_v[r, sl] + pe_v[r, sl]
            return 0

        lax.fori_loop(0, tok_per_worker, add_row, 0)
        pltpu.sync_copy(rows_v, out_hbm.at[pl.ds(base, tok_per_worker)])

    return body


def kernel(x, table):
    pe = _sinusoidal_pe(CTX, D)
    idx = x.reshape(-1).astype(jnp.int32)
    batch = idx.shape[0]
    tok_per_worker = batch // NUM_WORKERS
    tableT = table.T
    tail = jnp.pad(tableT[:, EDGE:], ((0, 0), (0, 128 - (VOCAB - EDGE))))
    out = _make_sc_kernel(batch, tok_per_worker)(idx, pe, tableT, tail)
    return out.reshape(x.shape + (D,))


# speedup vs baseline: 4.1201x; 1.1684x over previous
import functools

import jax
import jax.numpy as jnp
from jax import lax
from jax.experimental import pallas as pl
from jax.experimental.pallas import tpu as pltpu
from jax.experimental.pallas import tpu_sc as plsc

VOCAB = 1000000
CTX = 2048
D = 64

NUM_CORES = 2
NUM_SUBCORES = 16
NUM_WORKERS = NUM_CORES * NUM_SUBCORES
NBUF = 4
EDGE = (VOCAB // 128) * 128


def _sinusoidal_pe(context_size, embedding_size):
    positions = jnp.arange(context_size, dtype=jnp.float32)
    indices = jnp.arange(embedding_size // 2, dtype=jnp.float32)
    scaling_factor = 10000.0 ** (2.0 * indices / embedding_size)
    angles = positions[:, None] / scaling_factor
    pe = jnp.zeros((context_size, embedding_size), dtype=jnp.float32)
    pe = pe.at[:, 0::2].set(jnp.sin(angles))
    pe = pe.at[:, 1::2].set(jnp.cos(angles))
    return pe


def _make_sc_kernel(batch, tok_per_worker):
    mesh = plsc.VectorSubcoreMesh(
        core_axis_name="c",
        subcore_axis_name="s",
        num_cores=NUM_CORES,
        num_subcores=NUM_SUBCORES,
    )

    @functools.partial(
        pl.kernel,
        out_type=jax.ShapeDtypeStruct((batch, D), jnp.float32),
        mesh=mesh,
        scratch_types=[
            pltpu.VMEM((tok_per_worker + 16,), jnp.int32),
            pltpu.VMEM((tok_per_worker, D), jnp.float32),
            pltpu.VMEM((2 * NBUF + 1, D, 128), jnp.float32),
            pltpu.VMEM((2 * 2 * NBUF, D), jnp.float32),
            pltpu.SemaphoreType.DMA,
            pltpu.SemaphoreType.DMA,
            pltpu.SemaphoreType.DMA,
            pltpu.SemaphoreType.DMA,
        ],
        compiler_params=pltpu.CompilerParams(needs_layout_passes=False),
    )
    def body(
        idx_hbm, pe_hbm, tableT_hbm, tail_hbm, out_hbm,
        idx_s, pe_v, bufs, ring, sem_a, sem_b, sem_pe, sem_o,
    ):
        wid = lax.axis_index("s") * NUM_CORES + lax.axis_index("c")
        base = wid * tok_per_worker
        pe_base = lax.rem(base, CTX)
        pltpu.sync_copy(
            idx_hbm.at[pl.ds(base, tok_per_worker)],
            idx_s.at[pl.ds(0, tok_per_worker)],
        )
        pltpu.sync_copy(tail_hbm, bufs.at[2 * NBUF])
        pltpu.async_copy(
            pe_hbm.at[pl.ds(pe_base, tok_per_worker)], pe_v, sem_pe
        )
        iota16 = lax.iota(jnp.int32, 16)
        n_chunks = tok_per_worker // NBUF
        pair_rows = 2 * NBUF

        def fire(ch, gbase, gsem):
            vec = idx_s[pl.ds(ch * NBUF, 16)]
            for j in range(NBUF):
                off = jnp.minimum((vec[j] // 128) * 128, EDGE - 128)
                pltpu.async_copy(
                    tableT_hbm.at[:, pl.ds(off, 128)], bufs.at[gbase + j], gsem
                )

        def drain_extract(ch, gbase, gsem, slot):
            vec = idx_s[pl.ds(ch * NBUF, 16)]
            for j in range(NBUF):
                pltpu.make_async_copy(
                    tableT_hbm.at[:, pl.ds(0, 128)], bufs.at[gbase + j], gsem
                ).wait()
            for j in range(NBUF):
                edge = vec[j] >= EDGE
                off = jnp.minimum((vec[j] // 128) * 128, EDGE - 128)
                sel = jnp.full(
                    (16,), jnp.where(edge, 2 * NBUF, gbase + j), jnp.int32
                )
                col = jnp.where(edge, vec[j] - EDGE, vec[j] - off)
                voff = jnp.full((16,), col, jnp.int32)
                r = ch * NBUF + j
                rr = slot * pair_rows + gbase + j
                for c in range(D // 16):
                    sl = pl.ds(16 * c, 16)
                    g = plsc.load_gather(bufs, [sel, iota16 + (16 * c), voff])
                    ring[rr, sl] = g + pe_v[r, sl]

        fire(0, 0, sem_a)
        pltpu.make_async_copy(
            pe_hbm.at[pl.ds(0, tok_per_worker)], pe_v, sem_pe
        ).wait()

        def do_pair(it, _):
            ch_a = 2 * it
            slot = lax.rem(it, 2)

            @pl.when(it > 0)
            def _():
                pltpu.make_async_copy(
                    ring.at[pl.ds(0, pair_rows)],
                    out_hbm.at[pl.ds(base, pair_rows)],
                    sem_o,
                ).wait()

            fire(ch_a + 1, NBUF, sem_b)
            drain_extract(ch_a, 0, sem_a, slot)
            fire(jnp.minimum(ch_a + 2, n_chunks - 2), 0, sem_a)
            drain_extract(ch_a + 1, NBUF, sem_b, slot)
            pltpu.async_copy(
                ring.at[pl.ds(slot * pair_rows, pair_rows)],
                out_hbm.at[pl.ds(base + it * pair_rows, pair_rows)],
                sem_o,
            )
            return 0

        lax.fori_loop(0, n_chunks // 2, do_pair, 0)
        for j in range(NBUF):
            pltpu.make_async_copy(
                tableT_hbm.at[:, pl.ds(0, 128)], bufs.at[j], sem_a
            ).wait()
        pltpu.make_async_copy(
            ring.at[pl.ds(0, pair_rows)],
            out_hbm.at[pl.ds(base, pair_rows)],
            sem_o,
        ).wait()

    return body


def kernel(x, table):
    pe = _sinusoidal_pe(CTX, D)
    idx = x.reshape(-1).astype(jnp.int32)
    batch = idx.shape[0]
    tok_per_worker = batch // NUM_WORKERS
    tableT = table.T
    tail = jnp.pad(tableT[:, EDGE:], ((0, 0), (0, 128 - (VOCAB - EDGE))))
    out = _make_sc_kernel(batch, tok_per_worker)(idx, pe, tableT, tail)
    return out.reshape(x.shape + (D,))
